# Initial kernel scaffold; baseline (speedup 1.0000x reference)
import jax
import jax.numpy as jnp
from jax.experimental import pallas as pl


def kernel(x, tables):
    raise NotImplementedError("write your pallas kernel here")


# speedup vs baseline: 2.6422x; 2.6422x over previous
import numpy as np
import jax
import jax.numpy as jnp
from jax import lax
from jax.experimental import pallas as pl
from jax.experimental.pallas import tpu as pltpu
from jax.experimental.pallas import tpu_sc as plsc

_FIELD_DIMS = [3846] * 26
_F = 26
_D = 16
_B = 4096
_V = sum(_FIELD_DIMS)
_P = _F * (_F - 1) // 2

_NW = 32
_S = _B // _NW
_L = 16


def _worker(wid, tab, xiT, out, xiT_v, a_idx, b_idx, o_idx, o_base,
            a_rows, b_rows, sem_a, sem_b, sem_o):
    base = wid * _S

    pltpu.sync_copy(xiT.at[:, pl.ds(base, _S)], xiT_v)

    for m in range(_S // _L):
        sl = pl.ds(m * _L, _L)
        o_base[sl] = (lax.iota(jnp.int32, _L) + (base + m * _L)) * _P

    def pair_step(p, carry):
        i, j = carry
        iV = i * _V
        jV = j * _V
        for m in range(_S // _L):
            sl = pl.ds(m * _L, _L)
            a_idx[sl] = xiT_v[i, sl] + jV
            b_idx[sl] = xiT_v[j, sl] + iV
            o_idx[sl] = o_base[sl] + p
        cpa = pltpu.async_copy(tab.at[a_idx], a_rows, sem_a)
        cpb = pltpu.async_copy(tab.at[b_idx], b_rows, sem_b)
        cpa.wait()
        cpb.wait()
        for m in range(_S):
            a_rows[m] = a_rows[m] * b_rows[m]
        cpo = pltpu.async_copy(a_rows, out.at[o_idx], sem_o)
        cpo.wait()
        nj = j + 1
        wrap = nj >= _F
        ni = jnp.where(wrap, i + 1, i)
        nj = jnp.where(wrap, ni + 1, nj)
        return ni, nj

    lax.fori_loop(0, _P, pair_step, (jnp.int32(0), jnp.int32(1)))


def _ffm_body(tab, xiT, out, *rest):
    wid = lax.axis_index("s") * 2 + lax.axis_index("c")
    _worker(wid, tab, xiT, out, *rest)


def kernel(x, tables):
    offsets = np.concatenate([[0], np.cumsum(_FIELD_DIMS)[:-1]]).astype(np.int32)
    xiT = (x + jnp.asarray(offsets)[None, :]).T
    flat = tables.reshape(_F * _V, _D)
    mesh = plsc.VectorSubcoreMesh(core_axis_name="c", subcore_axis_name="s")
    run = pl.kernel(
        _ffm_body,
        out_type=jax.ShapeDtypeStruct((_B * _P, _D), jnp.float32),
        mesh=mesh,
        compiler_params=pltpu.CompilerParams(use_tc_tiling_on_sc=False),
        scratch_types=[
            pltpu.VMEM((_F, _S), jnp.int32),
            pltpu.VMEM((_S,), jnp.int32),
            pltpu.VMEM((_S,), jnp.int32),
            pltpu.VMEM((_S,), jnp.int32),
            pltpu.VMEM((_S,), jnp.int32),
            pltpu.VMEM((_S, _D), jnp.float32),
            pltpu.VMEM((_S, _D), jnp.float32),
            pltpu.SemaphoreType.DMA,
            pltpu.SemaphoreType.DMA,
            pltpu.SemaphoreType.DMA,
        ],
    )
    out = run(flat, xiT)
    return out.reshape(_B, _P, _D)


# speedup vs baseline: 2.7303x; 1.0334x over previous
import numpy as np
import jax
import jax.numpy as jnp
from jax import lax
from jax.experimental import pallas as pl
from jax.experimental.pallas import tpu as pltpu
from jax.experimental.pallas import tpu_sc as plsc

_FIELD_DIMS = [3846] * 26
_F = 26
_D = 16
_B = 4096
_V = sum(_FIELD_DIMS)
_P = _F * (_F - 1) // 2

_NW = 32
_S = _B // _NW
_L = 16
_NB = 5
_STEPS = _P // _NB


def _pairs():
    return [(i, j) for i in range(_F - 1) for j in range(i + 1, _F)]


def _worker(wid, tab, xiT, out, xiT_v, a_idx, b_idx, o_idx, o_base,
            a_rows, b_rows, o_rows, sem_g, sem_o):
    base = wid * _S

    pltpu.sync_copy(xiT.at[:, pl.ds(base, _S)], xiT_v)

    for m in range(_S // _L):
        sl = pl.ds(m * _L, _L)
        o_base[sl] = (lax.iota(jnp.int32, _L) + (base + m * _L)) * _P

    def build_gidx(slot, i, j):
        iV = i * _V
        jV = j * _V
        for m in range(_S // _L):
            sl = pl.ds(m * _L, _L)
            a_idx[slot, sl] = xiT_v[i, sl] + jV
            b_idx[slot, sl] = xiT_v[j, sl] + iV

    def fire_gathers(slot):
        pltpu.async_copy(tab.at[a_idx.at[slot]], a_rows.at[slot], sem_g.at[slot])
        pltpu.async_copy(tab.at[b_idx.at[slot]], b_rows.at[slot], sem_g.at[slot])

    def wait_gathers(slot):
        pltpu.make_async_copy(
            tab.at[a_idx.at[slot]], a_rows.at[slot], sem_g.at[slot]).wait()
        pltpu.make_async_copy(
            tab.at[b_idx.at[slot]], b_rows.at[slot], sem_g.at[slot]).wait()

    def wait_scatter(slot):
        pltpu.make_async_copy(
            o_rows.at[slot], out.at[o_idx.at[slot]], sem_o.at[slot]).wait()

    pair_tab = _pairs()
    for q in range(_NB - 1):
        build_gidx(q, pair_tab[q][0], pair_tab[q][1])
        fire_gathers(q)

    def step_fn(step, carry):
        if_, jf = carry
        for q in range(_NB):
            p = step * _NB + q
            slot = q
            fslot = (q + _NB - 1) % _NB
            fp = p + _NB - 1

            @pl.when(fp < _P)
            def _():
                build_gidx(fslot, if_, jf)
                fire_gathers(fslot)

            wait_gathers(slot)

            @pl.when(step >= 1)
            def _():
                wait_scatter(slot)

            for m in range(_S // _L):
                sl = pl.ds(m * _L, _L)
                o_idx[slot, sl] = o_base[sl] + p
            for m in range(_S):
                o_rows[slot, m] = a_rows[slot, m] * b_rows[slot, m]
            pltpu.async_copy(
                o_rows.at[slot], out.at[o_idx.at[slot]], sem_o.at[slot])

            jf2 = jf + 1
            wf = jf2 >= _F
            if_ = jnp.minimum(jnp.where(wf, if_ + 1, if_), _F - 2)
            jf = jnp.minimum(jnp.where(wf, if_ + 1, jf2), _F - 1)
        return if_, jf

    i0, j0 = pair_tab[_NB - 1]
    lax.fori_loop(0, _STEPS, step_fn, (jnp.int32(i0), jnp.int32(j0)))

    for q in range(_NB):
        wait_scatter(q)


def _ffm_body(tab, xiT, out, *rest):
    wid = lax.axis_index("s") * 2 + lax.axis_index("c")
    _worker(wid, tab, xiT, out, *rest)


def kernel(x, tables):
    offsets = np.concatenate([[0], np.cumsum(_FIELD_DIMS)[:-1]]).astype(np.int32)
    xiT = (x + jnp.asarray(offsets)[None, :]).T
    flat = tables.reshape(_F * _V, _D)
    mesh = plsc.VectorSubcoreMesh(core_axis_name="c", subcore_axis_name="s")
    run = pl.kernel(
        _ffm_body,
        out_type=jax.ShapeDtypeStruct((_B * _P, _D), jnp.float32),
        mesh=mesh,
        compiler_params=pltpu.CompilerParams(use_tc_tiling_on_sc=False),
        scratch_types=[
            pltpu.VMEM((_F, _S), jnp.int32),
            pltpu.VMEM((_NB, _S), jnp.int32),
            pltpu.VMEM((_NB, _S), jnp.int32),
            pltpu.VMEM((_NB, _S), jnp.int32),
            pltpu.VMEM((_S,), jnp.int32),
            pltpu.VMEM((_NB, _S, _D), jnp.float32),
            pltpu.VMEM((_NB, _S, _D), jnp.float32),
            pltpu.VMEM((_NB, _S, _D), jnp.float32),
            pltpu.SemaphoreType.DMA((_NB,)),
            pltpu.SemaphoreType.DMA((_NB,)),
        ],
    )
    out = run(flat, xiT)
    return out.reshape(_B, _P, _D)


# speedup vs baseline: 53.1282x; 19.4586x over previous
import numpy as np
import jax
import jax.numpy as jnp
from jax import lax
from jax.experimental import pallas as pl
from jax.experimental.pallas import tpu as pltpu
from jax.experimental.pallas import tpu_sc as plsc

_FD = 3846
_F = 26
_D = 16
_B = 4096
_V = _F * _FD
_P = _F * (_F - 1) // 2

_NW = 32
_L = 16
_DQ = 8
_UPP = _D // _DQ
_NU = _P * _UPP
_BW = 4096
_VPAD = ((_V + 127) >> 7) << 7
_CMAX = _VPAD - _BW
_NG = _B // _L


def _worker(wid, tabT, xT, out, bandA, bandB, xcolA, xcolB, stage):
    lo = (wid * _NU) >> 5
    hi = ((wid + 1) * _NU) >> 5

    p0 = lo >> 1
    dq0 = lo & 1

    def scan_i(t, acc):
        return acc + jnp.where((t * (51 - t)) // 2 <= p0, 1, 0)

    i0 = lax.fori_loop(1, _F, scan_i, jnp.int32(0))
    j0 = p0 - (i0 * (51 - i0)) // 2 + i0 + 1

    def unit_step(u, carry):
        i, j, dq = carry
        p = u >> 1
        offA = i * _FD
        offB = j * _FD
        cA = jnp.minimum((offA >> 7) << 7, _CMAX)
        cB = jnp.minimum((offB >> 7) << 7, _CMAX)
        shA = offA - cA
        shB = offB - cB
        cA = pl.multiple_of(cA, 128)
        cB = pl.multiple_of(cB, 128)
        rA = pl.multiple_of(j * _D + dq * _DQ, 8)
        rB = pl.multiple_of(i * _D + dq * _DQ, 8)

        pltpu.sync_copy(tabT.at[pl.ds(rA, _DQ), pl.ds(cA, _BW)], bandA)
        pltpu.sync_copy(tabT.at[pl.ds(rB, _DQ), pl.ds(cB, _BW)], bandB)
        pltpu.sync_copy(xT.at[i], xcolA)
        pltpu.sync_copy(xT.at[j], xcolB)

        def group(g, _):
            sl = pl.ds(g * _L, _L)
            ia = xcolA[sl] + shA
            ib = xcolB[sl] + shB
            for d in range(_DQ):
                dv = jnp.full((_L,), d, jnp.int32)
                a = plsc.load_gather(bandA, [dv, ia])
                b = plsc.load_gather(bandB, [dv, ib])
                stage[d, sl] = a * b
            return 0

        lax.fori_loop(0, _NG, group, 0)

        pltpu.sync_copy(stage, out.at[p, pl.ds(dq * _DQ, _DQ), :])

        dq2 = dq + 1
        wd = dq2 >= _UPP
        dq2 = jnp.where(wd, 0, dq2)
        j2 = jnp.where(wd, j + 1, j)
        wj = j2 >= _F
        i2 = jnp.where(wj, i + 1, i)
        j2 = jnp.where(wj, i + 2, j2)
        return i2, j2, dq2

    lax.fori_loop(lo, hi, unit_step, (i0, j0, jnp.int32(dq0)))


def _ffm_body(tabT, xT, out, *rest):
    wid = lax.axis_index("s") * 2 + lax.axis_index("c")
    _worker(wid, tabT, xT, out, *rest)


def kernel(x, tables):
    tabT = tables.transpose(0, 2, 1).reshape(_F * _D, _V)
    xT = x.T
    mesh = plsc.VectorSubcoreMesh(core_axis_name="c", subcore_axis_name="s")
    run = pl.kernel(
        _ffm_body,
        out_type=jax.ShapeDtypeStruct((_P, _D, _B), jnp.float32),
        mesh=mesh,
        compiler_params=pltpu.CompilerParams(needs_layout_passes=False),
        scratch_types=[
            pltpu.VMEM((_DQ, _BW), jnp.float32),
            pltpu.VMEM((_DQ, _BW), jnp.float32),
            pltpu.VMEM((_B,), jnp.int32),
            pltpu.VMEM((_B,), jnp.int32),
            pltpu.VMEM((_DQ, _B), jnp.float32),
        ],
    )
    out = run(tabT, xT)
    return out.transpose(2, 0, 1)


# speedup vs baseline: 61.3405x; 1.1546x over previous
import numpy as np
import jax
import jax.numpy as jnp
from jax import lax
from jax.experimental import pallas as pl
from jax.experimental.pallas import tpu as pltpu
from jax.experimental.pallas import tpu_sc as plsc

_FD = 3846
_F = 26
_D = 16
_B = 4096
_V = _F * _FD
_P = _F * (_F - 1) // 2

_NW = 32
_L = 16
_DQ = 8
_UPP = _D // _DQ
_NU = _P * _UPP
_BW = 4096
_VPAD = ((_V + 127) >> 7) << 7
_CMAX = _VPAD - _BW
_NG = _B // _L


_CH = 1024
_NCH = _B // _CH
_GPC = _CH // _L


def _worker(wid, tabT, xT, out, bandA, bandB, xcolA, xcolB, stage0, stage1,
            sem_a, sem_b, sem_x, sem_o):
    lo = (wid * _NU) >> 5
    hi = ((wid + 1) * _NU) >> 5

    p0 = lo >> 1
    dq0 = lo & 1

    def scan_i(t, acc):
        return acc + jnp.where((t * (51 - t)) // 2 <= p0, 1, 0)

    i0 = lax.fori_loop(1, _F, scan_i, jnp.int32(0))
    j0 = p0 - (i0 * (51 - i0)) // 2 + i0 + 1

    def unit_step(u, carry):
        i, j, dq = carry
        p = u >> 1
        offA = i * _FD
        offB = j * _FD
        cA = jnp.minimum((offA >> 7) << 7, _CMAX)
        cB = jnp.minimum((offB >> 7) << 7, _CMAX)
        shA = offA - cA
        shB = offB - cB
        cA = pl.multiple_of(cA, 128)
        cB = pl.multiple_of(cB, 128)
        rA = pl.multiple_of(j * _D + dq * _DQ, 8)
        rB = pl.multiple_of(i * _D + dq * _DQ, 8)

        cpa = pltpu.async_copy(tabT.at[pl.ds(rA, _DQ), pl.ds(cA, _BW)],
                               bandA, sem_a)
        cpb = pltpu.async_copy(tabT.at[pl.ds(rB, _DQ), pl.ds(cB, _BW)],
                               bandB, sem_b)
        cpx1 = pltpu.async_copy(xT.at[i], xcolA, sem_x)
        cpx2 = pltpu.async_copy(xT.at[j], xcolB, sem_x)
        cpa.wait()
        cpb.wait()
        cpx1.wait()
        cpx2.wait()

        dslice = pl.ds(dq * _DQ, _DQ)
        for c in range(_NCH):
            stage = (stage0, stage1)[c & 1]
            if c >= 2:
                pltpu.make_async_copy(
                    stage, out.at[p, dslice, pl.ds(c * _CH, _CH)],
                    sem_o).wait()
            else:
                @pl.when(u > lo)
                def _():
                    pltpu.make_async_copy(
                        stage, out.at[p, dslice, pl.ds(c * _CH, _CH)],
                        sem_o).wait()

            def group(g, _):
                gsl = pl.ds(g * _L, _L)
                xsl = pl.ds(c * _CH + g * _L, _L)
                ia = xcolA[xsl] + shA
                ib = xcolB[xsl] + shB
                for d in range(_DQ):
                    dv = jnp.full((_L,), d, jnp.int32)
                    a = plsc.load_gather(bandA, [dv, ia])
                    b = plsc.load_gather(bandB, [dv, ib])
                    stage[d, gsl] = a * b
                return 0

            lax.fori_loop(0, _GPC, group, 0)
            pltpu.async_copy(stage, out.at[p, dslice, pl.ds(c * _CH, _CH)],
                             sem_o)

        dq2 = dq + 1
        wd = dq2 >= _UPP
        dq2 = jnp.where(wd, 0, dq2)
        j2 = jnp.where(wd, j + 1, j)
        wj = j2 >= _F
        i2 = jnp.where(wj, i + 1, i)
        j2 = jnp.where(wj, i + 2, j2)
        return i2, j2, dq2

    lax.fori_loop(lo, hi, unit_step, (i0, j0, jnp.int32(dq0)))

    for s in (stage0, stage1):
        pltpu.make_async_copy(
            s, out.at[0, pl.ds(0, _DQ), pl.ds(0, _CH)], sem_o).wait()


def _ffm_body(tabT, xT, out, *rest):
    wid = lax.axis_index("s") * 2 + lax.axis_index("c")
    _worker(wid, tabT, xT, out, *rest)


def kernel(x, tables):
    tabT = tables.transpose(0, 2, 1).reshape(_F * _D, _V)
    xT = x.T
    mesh = plsc.VectorSubcoreMesh(core_axis_name="c", subcore_axis_name="s")
    run = pl.kernel(
        _ffm_body,
        out_type=jax.ShapeDtypeStruct((_P, _D, _B), jnp.float32),
        mesh=mesh,
        compiler_params=pltpu.CompilerParams(needs_layout_passes=False),
        scratch_types=[
            pltpu.VMEM((_DQ, _BW), jnp.float32),
            pltpu.VMEM((_DQ, _BW), jnp.float32),
            pltpu.VMEM((_B,), jnp.int32),
            pltpu.VMEM((_B,), jnp.int32),
            pltpu.VMEM((_DQ, _CH), jnp.float32),
            pltpu.VMEM((_DQ, _CH), jnp.float32),
            pltpu.SemaphoreType.DMA,
            pltpu.SemaphoreType.DMA,
            pltpu.SemaphoreType.DMA,
            pltpu.SemaphoreType.DMA,
        ],
    )
    out = run(tabT, xT)
    return out.transpose(2, 0, 1)


# speedup vs baseline: 129.8735x; 2.1173x over previous
import numpy as np
import jax
import jax.numpy as jnp
from jax import lax
from jax.experimental import pallas as pl
from jax.experimental.pallas import tpu as pltpu
from jax.experimental.pallas import tpu_sc as plsc

_FD = 3846
_F = 26
_D = 16
_B = 4096
_V = _F * _FD
_P = _F * (_F - 1) // 2

_NW = 32
_L = 16
_DQ = 8
_UPP = _D // _DQ
_NU = _P * _UPP
_BW = 4096
_VPAD = ((_V + 127) >> 7) << 7
_CMAX = _VPAD - _BW
_NG = _B // _L


_CH = 1024
_NCH = _B // _CH
_GPC = _CH // _L


def _worker(wid, tabT, xT, out, bandA, bandB, xcolA, xcolB, stage0, stage1,
            sem_a, sem_b, sem_x, sem_o):
    lo = (wid * _NU) >> 5
    hi = ((wid + 1) * _NU) >> 5

    p0 = lo >> 1
    dq0 = lo & 1

    def scan_i(t, acc):
        return acc + jnp.where((t * (51 - t)) // 2 <= p0, 1, 0)

    i0 = lax.fori_loop(1, _F, scan_i, jnp.int32(0))
    j0 = p0 - (i0 * (51 - i0)) // 2 + i0 + 1

    def unit_step(u, carry):
        i, j, dq = carry
        p = u >> 1
        offA = i * _FD
        offB = j * _FD
        cA = jnp.minimum((offA >> 7) << 7, _CMAX)
        cB = jnp.minimum((offB >> 7) << 7, _CMAX)
        shA = offA - cA
        shB = offB - cB
        cA = pl.multiple_of(cA, 128)
        cB = pl.multiple_of(cB, 128)
        rA = pl.multiple_of(j * _D + dq * _DQ, 8)
        rB = pl.multiple_of(i * _D + dq * _DQ, 8)

        cpa = pltpu.async_copy(tabT.at[pl.ds(rA, _DQ), pl.ds(cA, _BW)],
                               bandA, sem_a)
        cpb = pltpu.async_copy(tabT.at[pl.ds(rB, _DQ), pl.ds(cB, _BW)],
                               bandB, sem_b)
        cpx1 = pltpu.async_copy(xT.at[i], xcolA, sem_x)
        cpx2 = pltpu.async_copy(xT.at[j], xcolB, sem_x)
        cpa.wait()
        cpb.wait()
        cpx1.wait()
        cpx2.wait()

        dslice = pl.ds(dq * _DQ, _DQ)
        for c in range(_NCH):
            stage = (stage0, stage1)[c & 1]
            if c >= 2:
                pltpu.make_async_copy(
                    stage, out.at[p, dslice, pl.ds(c * _CH, _CH)],
                    sem_o).wait()
            else:
                @pl.when(u > lo)
                def _():
                    pltpu.make_async_copy(
                        stage, out.at[p, dslice, pl.ds(c * _CH, _CH)],
                        sem_o).wait()

            @plsc.parallel_loop(0, _GPC, 1, unroll=4)
            def group(g):
                gsl = pl.ds(g * _L, _L)
                xsl = pl.ds(c * _CH + g * _L, _L)
                ia = xcolA[xsl] + shA
                ib = xcolB[xsl] + shB
                for d in range(_DQ):
                    dv = jnp.full((_L,), d, jnp.int32)
                    a = plsc.load_gather(bandA, [dv, ia])
                    b = plsc.load_gather(bandB, [dv, ib])
                    stage[d, gsl] = a * b
            pltpu.async_copy(stage, out.at[p, dslice, pl.ds(c * _CH, _CH)],
                             sem_o)

        dq2 = dq + 1
        wd = dq2 >= _UPP
        dq2 = jnp.where(wd, 0, dq2)
        j2 = jnp.where(wd, j + 1, j)
        wj = j2 >= _F
        i2 = jnp.where(wj, i + 1, i)
        j2 = jnp.where(wj, i + 2, j2)
        return i2, j2, dq2

    lax.fori_loop(lo, hi, unit_step, (i0, j0, jnp.int32(dq0)))

    for s in (stage0, stage1):
        pltpu.make_async_copy(
            s, out.at[0, pl.ds(0, _DQ), pl.ds(0, _CH)], sem_o).wait()


def _ffm_body(tabT, xT, out, *rest):
    wid = lax.axis_index("s") * 2 + lax.axis_index("c")
    _worker(wid, tabT, xT, out, *rest)


def kernel(x, tables):
    tabT = tables.transpose(0, 2, 1).reshape(_F * _D, _V)
    xT = x.T
    mesh = plsc.VectorSubcoreMesh(core_axis_name="c", subcore_axis_name="s")
    run = pl.kernel(
        _ffm_body,
        out_type=jax.ShapeDtypeStruct((_P, _D, _B), jnp.float32),
        mesh=mesh,
        compiler_params=pltpu.CompilerParams(needs_layout_passes=False),
        scratch_types=[
            pltpu.VMEM((_DQ, _BW), jnp.float32),
            pltpu.VMEM((_DQ, _BW), jnp.float32),
            pltpu.VMEM((_B,), jnp.int32),
            pltpu.VMEM((_B,), jnp.int32),
            pltpu.VMEM((_DQ, _CH), jnp.float32),
            pltpu.VMEM((_DQ, _CH), jnp.float32),
            pltpu.SemaphoreType.DMA,
            pltpu.SemaphoreType.DMA,
            pltpu.SemaphoreType.DMA,
            pltpu.SemaphoreType.DMA,
        ],
    )
    out = run(tabT, xT)
    return out.transpose(2, 0, 1)
